# baseline (speedup 1.0000x reference)
import functools

import jax
import jax.numpy as jnp
from jax import lax
from jax.experimental import pallas as pl
from jax.experimental.pallas import tpu as pltpu
from jax.experimental.pallas import tpu_sc as plsc

N = 100000
D = 128
S = 1024
L = 16
NC = 2
NS = 16
NW = NC * NS
ACC = S * L
ROWS_W = 3136
LAST = N - (NW - 1) * ROWS_W

_mesh = plsc.VectorSubcoreMesh(core_axis_name="c", subcore_axis_name="s")
_params = pltpu.CompilerParams(needs_layout_passes=False)


def _wid():
    return lax.axis_index("s") * NC + lax.axis_index("c")


RB = 4000


def _rowdot_body(x_ref, w_ref, y_ref):
    y_ref[...] = lax.dot_general(
        x_ref[...], w_ref[...], (((1,), (0,)), ((), ())),
        preferred_element_type=jnp.float32)


_rowdot = pl.pallas_call(
    _rowdot_body,
    grid=(N // RB,),
    in_specs=[
        pl.BlockSpec((RB, D), lambda i: (i, 0)),
        pl.BlockSpec((D, 1), lambda i: (0, 0)),
    ],
    out_specs=pl.BlockSpec((RB, 1), lambda i: (i, 0)),
    out_shape=jax.ShapeDtypeStruct((N, 1), jnp.float32),
)


@functools.partial(
    pl.kernel,
    mesh=_mesh,
    out_type=[
        jax.ShapeDtypeStruct((NW * ACC,), jnp.float32),
        jax.ShapeDtypeStruct((NW * ACC,), jnp.float32),
    ],
    scratch_types=[
        pltpu.VMEM((ROWS_W,), jnp.float32),
        pltpu.VMEM((ROWS_W,), jnp.int32),
        pltpu.VMEM((ACC,), jnp.float32),
        pltpu.VMEM((ACC,), jnp.float32),
    ],
    compiler_params=_params,
)
def _segsum(y_hbm, b_hbm, a_hbm, c_hbm, ybuf, bbuf, acc, cnt):
    wid = _wid()
    iota = lax.iota(jnp.int32, L)
    zero16 = jnp.zeros((L,), jnp.float32)
    ones16 = jnp.ones((L,), jnp.float32)
    base = wid * ROWS_W

    @pl.when(wid < NW - 1)
    def _():
        pltpu.sync_copy(y_hbm.at[pl.ds(base, ROWS_W)], ybuf)
        pltpu.sync_copy(b_hbm.at[pl.ds(base, ROWS_W)], bbuf)

    @pl.when(wid == NW - 1)
    def _():
        pltpu.sync_copy(y_hbm.at[pl.ds(base, LAST)], ybuf.at[pl.ds(0, LAST)])
        pltpu.sync_copy(b_hbm.at[pl.ds(base, LAST)], bbuf.at[pl.ds(0, LAST)])

    def zbody(i, _):
        acc[pl.ds(i * 16, 16)] = zero16
        cnt[pl.ds(i * 16, 16)] = zero16
        return 0
    lax.fori_loop(0, S, zbody, 0)

    def gbody(g, _):
        for u in range(2):
            r0 = (g * 2 + u) * L
            yv = ybuf[pl.ds(r0, 16)]
            bv = bbuf[pl.ds(r0, 16)]
            idx = bv * 16 + iota
            plsc.addupdate_scatter(acc, [idx], yv)
            plsc.addupdate_scatter(cnt, [idx], ones16)
        return 0
    npairs = jnp.where(wid == NW - 1, LAST // (2 * L), ROWS_W // (2 * L))
    lax.fori_loop(0, npairs, gbody, 0)

    pltpu.sync_copy(acc, a_hbm.at[pl.ds(wid * ACC, ACC)])
    pltpu.sync_copy(cnt, c_hbm.at[pl.ds(wid * ACC, ACC)])


SEGW = S // NW
WIN = SEGW * L


@functools.partial(
    pl.kernel,
    mesh=_mesh,
    out_type=jax.ShapeDtypeStruct((S,), jnp.float32),
    scratch_types=[
        pltpu.VMEM((NW * WIN,), jnp.float32),
        pltpu.VMEM((NW * WIN,), jnp.float32),
        pltpu.VMEM((L,), jnp.float32),
        pltpu.VMEM((SEGW,), jnp.float32),
        pltpu.SemaphoreType.DMA,
    ],
    compiler_params=_params,
)
def _phase2(a_hbm, c_hbm, bias_hbm, out_hbm, wa, wc, bbuf, outv, sem):
    wid = _wid()
    sb = wid * WIN

    for half in range(4):
        for j in range(16):
            p = (half * 16 + j) % NW
            if half < 2:
                pltpu.async_copy(a_hbm.at[pl.ds(p * ACC + sb, WIN)],
                                 wa.at[pl.ds(p * WIN, WIN)], sem)
            else:
                pltpu.async_copy(c_hbm.at[pl.ds(p * ACC + sb, WIN)],
                                 wc.at[pl.ds(p * WIN, WIN)], sem)
        for j in range(16):
            p = (half * 16 + j) % NW
            if half < 2:
                pltpu.make_async_copy(a_hbm.at[pl.ds(0, WIN)],
                                      wa.at[pl.ds(p * WIN, WIN)], sem).wait()
            else:
                pltpu.make_async_copy(c_hbm.at[pl.ds(0, WIN)],
                                      wc.at[pl.ds(p * WIN, WIN)], sem).wait()

    pltpu.sync_copy(bias_hbm, bbuf)
    bv = bbuf[pl.ds(0, 16)]
    iota16 = lax.iota(jnp.int32, L) * 16

    def vbody(v, _):
        off = v * 16
        sa = wa[pl.ds(off, 16)]
        sc = wc[pl.ds(off, 16)]
        for p in range(1, NW):
            sa = sa + wa[pl.ds(p * WIN + off, 16)]
            sc = sc + wc[pl.ds(p * WIN + off, 16)]
        wa[pl.ds(off, 16)] = sa
        wc[pl.ds(off, 16)] = sc
        return 0
    lax.fori_loop(0, SEGW, vbody, 0)

    for g in range(SEGW // L):
        ta = jnp.zeros((L,), jnp.float32)
        tc = jnp.zeros((L,), jnp.float32)
        for l in range(L):
            idx = iota16 + (g * 256 + l)
            ta = ta + plsc.load_gather(wa, [idx])
            tc = tc + plsc.load_gather(wc, [idx])
        outv[pl.ds(g * 16, 16)] = ta / jnp.maximum(tc, 1.0) + bv
    pltpu.sync_copy(outv, out_hbm.at[pl.ds(wid * SEGW, SEGW)])


def kernel(x, batch, W, b):
    bi = batch.astype(jnp.int32)
    b16 = jnp.broadcast_to(b.astype(jnp.float32), (L,))
    y = _rowdot(x, W.reshape(D, 1).astype(jnp.float32)).reshape(N)
    a, c = _segsum(y, bi)
    return _phase2(a, c, b16)


# speedup vs baseline: 1.0100x; 1.0100x over previous
import functools

import jax
import jax.numpy as jnp
from jax import lax
from jax.experimental import pallas as pl
from jax.experimental.pallas import tpu as pltpu
from jax.experimental.pallas import tpu_sc as plsc

N = 100000
D = 128
S = 1024
L = 16
NC = 2
NS = 16
NW = NC * NS
ACC = S * L
ROWS_W = 3136
LAST = N - (NW - 1) * ROWS_W

_mesh = plsc.VectorSubcoreMesh(core_axis_name="c", subcore_axis_name="s")
_params = pltpu.CompilerParams(needs_layout_passes=False)


def _wid():
    return lax.axis_index("s") * NC + lax.axis_index("c")


RB = 4000


def _rowdot_body(x_ref, w_ref, y_ref):
    y_ref[...] = lax.dot_general(
        x_ref[...], w_ref[...], (((1,), (0,)), ((), ())),
        preferred_element_type=jnp.float32)


_rowdot = pl.pallas_call(
    _rowdot_body,
    grid=(N // RB,),
    in_specs=[
        pl.BlockSpec((RB, D), lambda i: (i, 0)),
        pl.BlockSpec((D, 1), lambda i: (0, 0)),
    ],
    out_specs=pl.BlockSpec((RB, 1), lambda i: (i, 0)),
    out_shape=jax.ShapeDtypeStruct((N, 1), jnp.float32),
)


@functools.partial(
    pl.kernel,
    mesh=_mesh,
    out_type=[
        jax.ShapeDtypeStruct((NW * ACC,), jnp.float32),
        jax.ShapeDtypeStruct((NW * ACC,), jnp.float32),
    ],
    scratch_types=[
        pltpu.VMEM((ROWS_W,), jnp.float32),
        pltpu.VMEM((ROWS_W,), jnp.int32),
        pltpu.VMEM((ACC,), jnp.float32),
        pltpu.VMEM((ACC,), jnp.float32),
        pltpu.SemaphoreType.DMA,
    ],
    compiler_params=_params,
)
def _segsum(y_hbm, b_hbm, a_hbm, c_hbm, ybuf, bbuf, acc, cnt, sem):
    wid = _wid()
    iota = lax.iota(jnp.int32, L)
    zero16 = jnp.zeros((L,), jnp.float32)
    ones16 = jnp.ones((L,), jnp.float32)
    base = wid * ROWS_W

    @pl.when(wid < NW - 1)
    def _():
        pltpu.sync_copy(y_hbm.at[pl.ds(base, ROWS_W)], ybuf)
        pltpu.sync_copy(b_hbm.at[pl.ds(base, ROWS_W)], bbuf)

    @pl.when(wid == NW - 1)
    def _():
        pltpu.sync_copy(y_hbm.at[pl.ds(base, LAST)], ybuf.at[pl.ds(0, LAST)])
        pltpu.sync_copy(b_hbm.at[pl.ds(base, LAST)], bbuf.at[pl.ds(0, LAST)])

    def zbody(i, _):
        acc[pl.ds(i * 16, 16)] = zero16
        cnt[pl.ds(i * 16, 16)] = zero16
        return 0
    lax.fori_loop(0, S, zbody, 0)

    iota_s = iota * S
    def gbody(g, _):
        for u in range(2):
            r0 = (g * 2 + u) * L
            yv = ybuf[pl.ds(r0, 16)]
            bv = bbuf[pl.ds(r0, 16)]
            idx = bv + iota_s
            plsc.addupdate_scatter(acc, [idx], yv)
            plsc.addupdate_scatter(cnt, [idx], ones16)
        return 0
    npairs = jnp.where(wid == NW - 1, LAST // (2 * L), ROWS_W // (2 * L))
    lax.fori_loop(0, npairs, gbody, 0)

    copy_a = pltpu.async_copy(acc, a_hbm.at[pl.ds(wid * ACC, ACC)], sem)
    copy_c = pltpu.async_copy(cnt, c_hbm.at[pl.ds(wid * ACC, ACC)], sem)
    copy_a.wait()
    copy_c.wait()


NP = NW * L


def _reduce_body(a_ref, c_ref, b_ref, o_ref):
    sums = jnp.sum(a_ref[...], axis=0)
    cnts = jnp.sum(c_ref[...], axis=0)
    o_ref[...] = sums / jnp.maximum(cnts, 1.0) + b_ref[...]


_reduce = pl.pallas_call(
    _reduce_body,
    in_specs=[
        pl.BlockSpec((NP, S), lambda: (0, 0)),
        pl.BlockSpec((NP, S), lambda: (0, 0)),
        pl.BlockSpec((S,), lambda: (0,)),
    ],
    out_specs=pl.BlockSpec((S,), lambda: (0,)),
    out_shape=jax.ShapeDtypeStruct((S,), jnp.float32),
)


def kernel(x, batch, W, b):
    bi = batch.astype(jnp.int32)
    bvec = jnp.broadcast_to(b.astype(jnp.float32), (S,))
    y = _rowdot(x, W.reshape(D, 1).astype(jnp.float32)).reshape(N)
    a, c = _segsum(y, bi)
    return _reduce(a.reshape(NP, S), c.reshape(NP, S), bvec)


# speedup vs baseline: 1.0718x; 1.0612x over previous
import functools

import jax
import jax.numpy as jnp
from jax import lax
from jax.experimental import pallas as pl
from jax.experimental.pallas import tpu as pltpu
from jax.experimental.pallas import tpu_sc as plsc

N = 100000
D = 128
S = 1024
L = 16
NC = 2
NS = 16
NW = NC * NS
ACC = S * L
ROWS_W = 3136
LAST = N - (NW - 1) * ROWS_W

_mesh = plsc.VectorSubcoreMesh(core_axis_name="c", subcore_axis_name="s")
_params = pltpu.CompilerParams(needs_layout_passes=False)


def _wid():
    return lax.axis_index("s") * NC + lax.axis_index("c")


RB = 10000


def _rowdot_body(x_ref, w_ref, y_ref):
    y_ref[...] = lax.dot_general(
        x_ref[...], w_ref[...], (((1,), (0,)), ((), ())),
        preferred_element_type=jnp.float32)


_rowdot = pl.pallas_call(
    _rowdot_body,
    grid=(N // RB,),
    in_specs=[
        pl.BlockSpec((RB, D), lambda i: (i, 0)),
        pl.BlockSpec((D, 1), lambda i: (0, 0)),
    ],
    out_specs=pl.BlockSpec((RB, 1), lambda i: (i, 0)),
    out_shape=jax.ShapeDtypeStruct((N, 1), jnp.float32),
    compiler_params=pltpu.CompilerParams(
        dimension_semantics=("parallel",)),
)


@functools.partial(
    pl.kernel,
    mesh=_mesh,
    out_type=[
        jax.ShapeDtypeStruct((NW * ACC,), jnp.float32),
        jax.ShapeDtypeStruct((NW * ACC,), jnp.float32),
    ],
    scratch_types=[
        pltpu.VMEM((ROWS_W,), jnp.float32),
        pltpu.VMEM((ROWS_W,), jnp.int32),
        pltpu.VMEM((ACC,), jnp.float32),
        pltpu.VMEM((ACC,), jnp.float32),
        pltpu.SemaphoreType.DMA,
    ],
    compiler_params=_params,
)
def _segsum(y_hbm, b_hbm, a_hbm, c_hbm, ybuf, bbuf, acc, cnt, sem):
    wid = _wid()
    iota = lax.iota(jnp.int32, L)
    zero16 = jnp.zeros((L,), jnp.float32)
    ones16 = jnp.ones((L,), jnp.float32)
    base = wid * ROWS_W

    @pl.when(wid < NW - 1)
    def _():
        pltpu.sync_copy(y_hbm.at[pl.ds(base, ROWS_W)], ybuf)
        pltpu.sync_copy(b_hbm.at[pl.ds(base, ROWS_W)], bbuf)

    @pl.when(wid == NW - 1)
    def _():
        pltpu.sync_copy(y_hbm.at[pl.ds(base, LAST)], ybuf.at[pl.ds(0, LAST)])
        pltpu.sync_copy(b_hbm.at[pl.ds(base, LAST)], bbuf.at[pl.ds(0, LAST)])

    def zbody(i, _):
        acc[pl.ds(i * 16, 16)] = zero16
        cnt[pl.ds(i * 16, 16)] = zero16
        return 0
    lax.fori_loop(0, S, zbody, 0)

    iota_s = iota * S
    def gbody(g, _):
        for u in range(2):
            r0 = (g * 2 + u) * L
            yv = ybuf[pl.ds(r0, 16)]
            bv = bbuf[pl.ds(r0, 16)]
            idx = bv + iota_s
            plsc.addupdate_scatter(acc, [idx], yv)
            plsc.addupdate_scatter(cnt, [idx], ones16)
        return 0
    npairs = jnp.where(wid == NW - 1, LAST // (2 * L), ROWS_W // (2 * L))
    lax.fori_loop(0, npairs, gbody, 0)

    copy_a = pltpu.async_copy(acc, a_hbm.at[pl.ds(wid * ACC, ACC)], sem)
    copy_c = pltpu.async_copy(cnt, c_hbm.at[pl.ds(wid * ACC, ACC)], sem)
    copy_a.wait()
    copy_c.wait()


NP = NW * L


def _reduce_body(a_ref, c_ref, b_ref, o_ref):
    sums = jnp.sum(a_ref[...], axis=0)
    cnts = jnp.sum(c_ref[...], axis=0)
    o_ref[...] = sums / jnp.maximum(cnts, 1.0) + b_ref[...]


_reduce = pl.pallas_call(
    _reduce_body,
    in_specs=[
        pl.BlockSpec((NP, S), lambda: (0, 0)),
        pl.BlockSpec((NP, S), lambda: (0, 0)),
        pl.BlockSpec((S,), lambda: (0,)),
    ],
    out_specs=pl.BlockSpec((S,), lambda: (0,)),
    out_shape=jax.ShapeDtypeStruct((S,), jnp.float32),
)


def kernel(x, batch, W, b):
    bi = batch.astype(jnp.int32)
    bvec = jnp.broadcast_to(b.astype(jnp.float32), (S,))
    y = _rowdot(x, W.reshape(D, 1).astype(jnp.float32)).reshape(N)
    a, c = _segsum(y, bi)
    return _reduce(a.reshape(NP, S), c.reshape(NP, S), bvec)
